# baseline (speedup 1.0000x reference)
import functools
import jax
import jax.numpy as jnp
from jax import lax
from jax.experimental import pallas as pl
from jax.experimental.pallas import tpu as pltpu
from jax.experimental.pallas import tpu_sc as plsc

_K = 8
_ROWS = 128
_COLS = 32768
_L = 16
_NC = 2
_NS = 16
_NW = _NC * _NS
_RPW = _ROWS // _NW
_VPR = _COLS // _L
_CHUNK = 16
_NCHUNK = _VPR // _CHUNK
_NEG = float("-inf")


def _sort16(x):
    r = plsc.sort_key_val(x, x, descending=True)
    return r[0] if isinstance(r, (tuple, list)) else r


def _merge16(C, x):
    xs = _sort16(x)
    y = jnp.maximum(xs, lax.rev(C, (0,)))
    return _sort16(y)


def _sc_body(in_hbm, out_hbm, buf0, buf1, cmax, hitlist, cand, stage,
             sem0, sem1):
    wid = lax.axis_index("s") * _NC + lax.axis_index("c")
    base = wid * _RPW
    bufs = [buf0, buf1]
    sems = [sem0, sem1]
    lane = lax.broadcasted_iota(jnp.int32, (_L,), 0)
    neg = jnp.full((_L,), _NEG, jnp.float32)
    zero_i = jnp.zeros((_L,), jnp.int32)
    copies = [None, None]

    _NSPLIT = 4
    _Q = _COLS // _NSPLIT

    def start(r):
        b = r % 2
        copies[b] = [
            pltpu.async_copy(in_hbm.at[base + r, pl.ds(q * _Q, _Q)],
                             bufs[b].at[pl.ds(q * _Q, _Q)], sems[b])
            for q in range(_NSPLIT)
        ]

    start(0)
    for r in range(_RPW):
        b = r % 2
        for cp in copies[b]:
            cp.wait()
        if r + 1 < _RPW:
            start(r + 1)
        buf = bufs[b]

        @plsc.parallel_loop(0, _NCHUNK, 1, unroll=4, carry=neg)
        def m(c, acc):
            off = c * (_CHUNK * _L)
            a0, a1, a2, a3 = neg, neg, neg, neg
            for j in range(0, _CHUNK, 4):
                a0 = jnp.maximum(a0, buf[pl.ds(off + j * _L, _L)])
                a1 = jnp.maximum(a1, buf[pl.ds(off + (j + 1) * _L, _L)])
                a2 = jnp.maximum(a2, buf[pl.ds(off + (j + 2) * _L, _L)])
                a3 = jnp.maximum(a3, buf[pl.ds(off + (j + 3) * _L, _L)])
            mc = jnp.maximum(jnp.maximum(a0, a1), jnp.maximum(a2, a3))
            cm = plsc.cummax(mc)
            plsc.store_scatter(cmax, [jnp.full((_L,), c, jnp.int32)], cm,
                               mask=lane == _L - 1)
            return jnp.maximum(acc, mc)

        tau = jnp.full((_L,), _sort16(m)[_K - 1], jnp.float32)

        n = zero_i
        for g in range(_NCHUNK // _L):
            hc = cmax[pl.ds(g * _L, _L)]
            hit = hc >= tau
            cs = plsc.cumsum(hit.astype(jnp.int32))
            plsc.store_scatter(hitlist, [n + cs - 1], g * _L + lane, mask=hit)
            n = n + plsc.all_reduce_population_count(hit)
        nh = n[0]

        def compact(i, nc):
            cid = hitlist[pl.ds(i, _L)][0]
            off = cid * (_CHUNK * _L)
            for j in range(_CHUNK):
                x = buf[pl.ds(off + j * _L, _L)]
                hit = x >= tau
                cs = plsc.cumsum(hit.astype(jnp.int32))
                plsc.store_scatter(cand, [nc + cs - 1], x, mask=hit)
                nc = nc + plsc.all_reduce_population_count(hit)
            return nc

        nc = lax.fori_loop(0, nh, compact, zero_i)
        plsc.store_scatter(cand, [nc + lane], neg)

        nv = (nc[0] + _L - 1) // _L

        def merge_step(i, C):
            return _merge16(C, cand[pl.ds(i * _L, _L)])

        C = lax.fori_loop(0, nv, merge_step, neg)

        plsc.store_compressed(stage.at[pl.ds(r * _K, _L)], C, mask=lane < _K)

    pltpu.sync_copy(stage.at[pl.ds(0, _RPW * _K)],
                    out_hbm.at[pl.ds(base * _K, _RPW * _K)])


def kernel(inputs):
    mesh = plsc.VectorSubcoreMesh(core_axis_name="c", subcore_axis_name="s",
                                  num_cores=_NC, num_subcores=_NS)
    out = pl.kernel(
        _sc_body,
        out_type=jax.ShapeDtypeStruct((_ROWS * _K,), jnp.float32),
        mesh=mesh,
        compiler_params=pltpu.CompilerParams(needs_layout_passes=False),
        scratch_types=[
            pltpu.VMEM((_COLS,), jnp.float32),
            pltpu.VMEM((_COLS,), jnp.float32),
            pltpu.VMEM((_NCHUNK,), jnp.float32),
            pltpu.VMEM((_NCHUNK + _L,), jnp.int32),
            pltpu.VMEM((_COLS + _L,), jnp.float32),
            pltpu.VMEM((48,), jnp.float32),
            pltpu.SemaphoreType.DMA,
            pltpu.SemaphoreType.DMA,
        ],
    )(inputs)
    return out.reshape(_ROWS, _K)


# speedup vs baseline: 1.0052x; 1.0052x over previous
import functools
import jax
import jax.numpy as jnp
from jax import lax
from jax.experimental import pallas as pl
from jax.experimental.pallas import tpu as pltpu
from jax.experimental.pallas import tpu_sc as plsc

_K = 8
_ROWS = 128
_COLS = 32768
_L = 16
_NC = 2
_NS = 16
_NW = _NC * _NS
_RPW = _ROWS // _NW
_VPR = _COLS // _L
_CHUNK = 16
_NCHUNK = _VPR // _CHUNK
_NEG = float("-inf")


def _sort16(x):
    r = plsc.sort_key_val(x, x, descending=True)
    return r[0] if isinstance(r, (tuple, list)) else r


def _merge16(C, x):
    xs = _sort16(x)
    y = jnp.maximum(xs, lax.rev(C, (0,)))
    return _sort16(y)


def _sc_body(in_hbm, out_hbm, buf0, buf1, cmax, hitlist, cand, stage,
             sem0, sem1):
    wid = lax.axis_index("s") * _NC + lax.axis_index("c")
    base = wid * _RPW
    bufs = [buf0, buf1]
    sems = [sem0, sem1]
    lane = lax.broadcasted_iota(jnp.int32, (_L,), 0)
    neg = jnp.full((_L,), _NEG, jnp.float32)
    zero_i = jnp.zeros((_L,), jnp.int32)
    copies = [None, None]

    def start(r):
        b = r % 2
        copies[b] = pltpu.async_copy(in_hbm.at[base + r], bufs[b], sems[b])

    start(0)
    for r in range(_RPW):
        b = r % 2
        copies[b].wait()
        if r + 1 < _RPW:
            start(r + 1)
        buf = bufs[b]

        @plsc.parallel_loop(0, _NCHUNK, 1, unroll=4, carry=neg)
        def m(c, acc):
            off = c * (_CHUNK * _L)
            a0, a1, a2, a3 = neg, neg, neg, neg
            for j in range(0, _CHUNK, 4):
                a0 = jnp.maximum(a0, buf[pl.ds(off + j * _L, _L)])
                a1 = jnp.maximum(a1, buf[pl.ds(off + (j + 1) * _L, _L)])
                a2 = jnp.maximum(a2, buf[pl.ds(off + (j + 2) * _L, _L)])
                a3 = jnp.maximum(a3, buf[pl.ds(off + (j + 3) * _L, _L)])
            mc = jnp.maximum(jnp.maximum(a0, a1), jnp.maximum(a2, a3))
            cm = plsc.cummax(mc)
            plsc.store_scatter(cmax, [jnp.full((_L,), c, jnp.int32)], cm,
                               mask=lane == _L - 1)
            return jnp.maximum(acc, mc)

        tau = jnp.full((_L,), _sort16(m)[_K - 1], jnp.float32)

        n = zero_i
        for g in range(_NCHUNK // _L):
            hc = cmax[pl.ds(g * _L, _L)]
            hit = hc >= tau
            cs = plsc.cumsum(hit.astype(jnp.int32))
            plsc.store_scatter(hitlist, [n + cs - 1], g * _L + lane, mask=hit)
            n = n + plsc.all_reduce_population_count(hit)
        nh = n[0]

        def compact(i, nc):
            cid = hitlist[pl.ds(i, _L)][0]
            off = cid * (_CHUNK * _L)
            for j in range(_CHUNK):
                x = buf[pl.ds(off + j * _L, _L)]
                hit = x >= tau
                cs = plsc.cumsum(hit.astype(jnp.int32))
                plsc.store_scatter(cand, [nc + cs - 1], x, mask=hit)
                nc = nc + plsc.all_reduce_population_count(hit)
            return nc

        nc = lax.fori_loop(0, nh, compact, zero_i)
        plsc.store_scatter(cand, [nc + lane], neg)

        nv = (nc[0] + _L - 1) // _L

        def merge_step(i, C):
            return _merge16(C, cand[pl.ds(i * _L, _L)])

        C = lax.fori_loop(0, nv, merge_step, neg)

        plsc.store_compressed(stage.at[pl.ds(r * _K, _L)], C, mask=lane < _K)

    pltpu.sync_copy(stage.at[pl.ds(0, _RPW * _K)],
                    out_hbm.at[pl.ds(base * _K, _RPW * _K)])


def kernel(inputs):
    mesh = plsc.VectorSubcoreMesh(core_axis_name="c", subcore_axis_name="s",
                                  num_cores=_NC, num_subcores=_NS)
    out = pl.kernel(
        _sc_body,
        out_type=jax.ShapeDtypeStruct((_ROWS * _K,), jnp.float32),
        mesh=mesh,
        compiler_params=pltpu.CompilerParams(needs_layout_passes=False),
        scratch_types=[
            pltpu.VMEM((_COLS,), jnp.float32),
            pltpu.VMEM((_COLS,), jnp.float32),
            pltpu.VMEM((_NCHUNK,), jnp.float32),
            pltpu.VMEM((_NCHUNK + _L,), jnp.int32),
            pltpu.VMEM((_COLS + _L,), jnp.float32),
            pltpu.VMEM((48,), jnp.float32),
            pltpu.SemaphoreType.DMA,
            pltpu.SemaphoreType.DMA,
        ],
    )(inputs)
    return out.reshape(_ROWS, _K)
